# Initial kernel scaffold; baseline (speedup 1.0000x reference)
import jax
import jax.numpy as jnp
from jax.experimental import pallas as pl


def kernel(attn_bias, spatial_pos, edge_input, attn_edge_type, edge_weight, spatial_weight, graph_token_weight, edge_dis_weight):
    raise NotImplementedError("write your pallas kernel here")


# speedup vs baseline: 20.7632x; 20.7632x over previous
import functools

import jax
import jax.numpy as jnp
from jax import lax
from jax.experimental import pallas as pl
from jax.experimental.pallas import tpu as pltpu
from jax.experimental.pallas import tpu_sc as plsc

B, N, H = 16, 128, 32
MHD, EF = 5, 3
NER = 513
NSP = 512
TBL_ROWS = MHD * NER + NSP
HP = H // 2
NC, NS, L = 2, 16, 16
NW = NC * NS
ROWS = B * N
ROWS_PER_W = ROWS // NW


def _prep_body(ew_ref, w3_ref, sw_ref, out_ref):
    ew = ew_ref[...]
    parts = [jnp.dot(ew, w3_ref[m], preferred_element_type=jnp.float32)
             for m in range(MHD)]
    parts.append(sw_ref[...])
    out_ref[...] = jnp.concatenate(parts, axis=0)


def _prep_table(edge_weight, w3, spatial_weight):
    return pl.pallas_call(
        _prep_body,
        out_shape=jax.ShapeDtypeStruct((TBL_ROWS, H), jnp.float32),
    )(edge_weight, w3, spatial_weight)


def _sc_body(tcat_hbm, sp_hbm, ei_hbm, out_hbm, tbl_v, ei_v, sp_v, r_v, sem):
    wid = lax.axis_index("s") * NC + lax.axis_index("c")
    pltpu.sync_copy(tcat_hbm, tbl_v)

    def row_body(k, carry):
        row = wid * ROWS_PER_W + k
        bb = lax.div(row, N)
        ii = lax.rem(row, N)
        pltpu.sync_copy(ei_hbm.at[row], ei_v)
        pltpu.sync_copy(sp_hbm.at[row], sp_v)
        for jc in range(N // L):
            sl = pl.ds(jc * L, L)
            spv = sp_v[sl]
            sp_f = jnp.clip(spv - 1, 1, MHD).astype(jnp.float32)
            inv_sp = 1.0 / sp_f
            idx = []
            wgt = []
            for m in range(MHD):
                im = [ei_v[pl.ds((m * EF + f) * N + jc * L, L)]
                      for f in range(EF)]
                cnt = (jnp.minimum(im[0], 1) + jnp.minimum(im[1], 1)
                       + jnp.minimum(im[2], 1))
                denom = jnp.maximum(cnt, 1).astype(jnp.float32)
                w = inv_sp / denom
                wgt.append(plsc.pack(w, w, format=plsc.PackFormat.INTERLEAVED))
                idx.append([(x + m * NER) * HP for x in im])
            sprow = (spv + MHD * NER) * HP

            def h_body(hp, carry2):
                vh = jnp.full((L,), hp, dtype=jnp.int32)
                acc = plsc.bitcast(plsc.load_gather(tbl_v, [sprow + vh]),
                                   jnp.bfloat16)
                for m in range(MHD):
                    g0 = plsc.bitcast(
                        plsc.load_gather(tbl_v, [idx[m][0] + vh]), jnp.bfloat16)
                    g1 = plsc.bitcast(
                        plsc.load_gather(tbl_v, [idx[m][1] + vh]), jnp.bfloat16)
                    g2 = plsc.bitcast(
                        plsc.load_gather(tbl_v, [idx[m][2] + vh]), jnp.bfloat16)
                    acc = acc + (g0 + g1 + g2) * wgt[m]
                ev, od = plsc.unpack(acc, format=plsc.PackFormat.INTERLEAVED)
                r_v[2 * hp, sl] = ev
                r_v[2 * hp + 1, sl] = od
                return carry2

            lax.fori_loop(0, HP, h_body, 0)
        pltpu.sync_copy(r_v, out_hbm.at[bb, :, ii, :])
        return carry

    lax.fori_loop(0, ROWS_PER_W, row_body, 0)


_sc_gather = functools.partial(
    pl.kernel,
    out_type=jax.ShapeDtypeStruct((B, H, N, N), jnp.float32),
    mesh=plsc.VectorSubcoreMesh(core_axis_name="c", subcore_axis_name="s"),
    compiler_params=pltpu.CompilerParams(needs_layout_passes=False),
    scratch_types=[
        pltpu.VMEM((TBL_ROWS * HP,), jnp.int32),
        pltpu.VMEM((MHD * EF * N,), jnp.int32),
        pltpu.VMEM((N,), jnp.int32),
        pltpu.VMEM((H, N), jnp.float32),
        pltpu.SemaphoreType.DMA,
    ],
)(_sc_body)


def _add_body(ab_ref, r_ref, gt_ref, out_ref):
    h = pl.program_id(1)
    t = gt_ref[0, h]
    ab = ab_ref[0, 0]
    rp = jnp.pad(r_ref[0, 0], ((1, 0), (1, 0)))
    i0 = lax.broadcasted_iota(jnp.int32, (N + 1, N + 1), 0)
    i1 = lax.broadcasted_iota(jnp.int32, (N + 1, N + 1), 1)
    edge_mask = jnp.logical_or(i0 == 0, i1 == 0)
    out_ref[0, 0] = ab + rp + jnp.where(edge_mask, t, 0.0)


def _add_bias(attn_bias, r4, graph_token_weight):
    return pl.pallas_call(
        _add_body,
        grid=(B, H),
        in_specs=[
            pl.BlockSpec((1, 1, N + 1, N + 1), lambda b, h: (b, h, 0, 0)),
            pl.BlockSpec((1, 1, N, N), lambda b, h: (b, h, 0, 0)),
            pl.BlockSpec(memory_space=pltpu.SMEM),
        ],
        out_specs=pl.BlockSpec((1, 1, N + 1, N + 1), lambda b, h: (b, h, 0, 0)),
        out_shape=jax.ShapeDtypeStruct((B, H, N + 1, N + 1), jnp.float32),
        compiler_params=pltpu.CompilerParams(
            dimension_semantics=("parallel", "parallel")),
    )(attn_bias, r4, graph_token_weight)


def kernel(attn_bias, spatial_pos, edge_input, attn_edge_type,
           edge_weight, spatial_weight, graph_token_weight, edge_dis_weight):
    del attn_edge_type
    w3 = edge_dis_weight.reshape(-1, H, H)[:MHD]
    tcat = _prep_table(edge_weight, w3, spatial_weight)
    packed = lax.bitcast_convert_type(
        tcat.astype(jnp.bfloat16).reshape(TBL_ROWS, HP, 2), jnp.int32)
    tflat = packed.reshape(TBL_ROWS * HP)
    sp_t = spatial_pos.reshape(ROWS, N)
    ei_t = jnp.transpose(edge_input.reshape(ROWS, N, MHD * EF),
                         (0, 2, 1)).reshape(ROWS, MHD * EF * N)
    r = _sc_gather(tflat, sp_t, ei_t)
    return _add_bias(attn_bias, r, graph_token_weight)
